# Initial kernel scaffold; baseline (speedup 1.0000x reference)
import jax
import jax.numpy as jnp
from jax.experimental import pallas as pl


def kernel(modality_tokens, timestamps, channel_embed, pos_embed, month_table):
    raise NotImplementedError("write your pallas kernel here")


# speedup vs baseline: 1.5230x; 1.5230x over previous
import jax
import jax.numpy as jnp
from jax.experimental import pallas as pl
from jax.experimental.pallas import tpu as pltpu

B, T, BS, EMBED = 64, 24, 8, 1024
N = EMBED // 4


def _body(ts_ref, tok_ref, ch_ref, pe_ref, mt_ref, out_ref):
    b = pl.program_id(0)
    ch = ch_ref[...]
    for t in range(T):
        ts = ts_ref[b, t]
        me = mt_ref[ts, :]
        pe = pe_ref[t, :]
        tok = tok_ref[0, t]
        out_ref[0, t] = jnp.concatenate(
            [
                tok[:, :N] + ch,
                tok[:, N:2 * N] + pe[None, :],
                tok[:, 2 * N:3 * N] + me[None, :],
                tok[:, 3 * N:],
            ],
            axis=-1,
        )


def kernel(modality_tokens, timestamps, channel_embed, pos_embed, month_table):
    ts32 = timestamps.astype(jnp.int32)
    grid_spec = pltpu.PrefetchScalarGridSpec(
        num_scalar_prefetch=1,
        grid=(B,),
        in_specs=[
            pl.BlockSpec((1, T, BS, EMBED), lambda b, ts: (b, 0, 0, 0)),
            pl.BlockSpec((BS, N), lambda b, ts: (0, 0)),
            pl.BlockSpec((T, N), lambda b, ts: (0, 0)),
            pl.BlockSpec((12, N), lambda b, ts: (0, 0)),
        ],
        out_specs=pl.BlockSpec((1, T, BS, EMBED), lambda b, ts: (b, 0, 0, 0)),
    )
    return pl.pallas_call(
        _body,
        grid_spec=grid_spec,
        out_shape=jax.ShapeDtypeStruct((B, T, BS, EMBED), jnp.float32),
    )(ts32, modality_tokens, channel_embed, pos_embed, month_table)


# speedup vs baseline: 2.4873x; 1.6331x over previous
import jax
import jax.numpy as jnp
from jax.experimental import pallas as pl
from jax.experimental.pallas import tpu as pltpu

B, T, BS, EMBED = 64, 24, 8, 1024
N = EMBED // 4


BBLK = 4


def _body(ts_ref, tok_ref, ch_ref, pe_ref, mt_ref, out_ref):
    bb = pl.program_id(0)
    ch = ch_ref[...]
    for bi in range(BBLK):
        b = bb * BBLK + bi
        for t in range(T):
            ts = ts_ref[b, t]
            me = mt_ref[ts, :]
            pe = pe_ref[t, :]
            tok = tok_ref[bi, t]
            out_ref[bi, t] = jnp.concatenate(
                [
                    tok[:, :N] + ch,
                    tok[:, N:2 * N] + pe[None, :],
                    tok[:, 2 * N:3 * N] + me[None, :],
                    tok[:, 3 * N:],
                ],
                axis=-1,
            )


def kernel(modality_tokens, timestamps, channel_embed, pos_embed, month_table):
    ts32 = timestamps.astype(jnp.int32)
    grid_spec = pltpu.PrefetchScalarGridSpec(
        num_scalar_prefetch=1,
        grid=(B // BBLK,),
        in_specs=[
            pl.BlockSpec((BBLK, T, BS, EMBED), lambda b, ts: (b, 0, 0, 0)),
            pl.BlockSpec((BS, N), lambda b, ts: (0, 0)),
            pl.BlockSpec((T, N), lambda b, ts: (0, 0)),
            pl.BlockSpec((12, N), lambda b, ts: (0, 0)),
        ],
        out_specs=pl.BlockSpec((BBLK, T, BS, EMBED), lambda b, ts: (b, 0, 0, 0)),
    )
    return pl.pallas_call(
        _body,
        grid_spec=grid_spec,
        out_shape=jax.ShapeDtypeStruct((B, T, BS, EMBED), jnp.float32),
    )(ts32, modality_tokens, channel_embed, pos_embed, month_table)


# speedup vs baseline: 2.6088x; 1.0488x over previous
import jax
import jax.numpy as jnp
from jax.experimental import pallas as pl
from jax.experimental.pallas import tpu as pltpu

B, T, BS, EMBED = 64, 24, 8, 1024
N = EMBED // 4


BBLK = 8


def _body(ts_ref, tok_ref, ch_ref, pe_ref, mt_ref, out_ref):
    bb = pl.program_id(0)
    ch = ch_ref[...]
    for bi in range(BBLK):
        b = bb * BBLK + bi
        for t in range(T):
            ts = ts_ref[b, t]
            me = mt_ref[ts, :]
            pe = pe_ref[t, :]
            tok = tok_ref[bi, t]
            out_ref[bi, t] = jnp.concatenate(
                [
                    tok[:, :N] + ch,
                    tok[:, N:2 * N] + pe[None, :],
                    tok[:, 2 * N:3 * N] + me[None, :],
                    tok[:, 3 * N:],
                ],
                axis=-1,
            )


def kernel(modality_tokens, timestamps, channel_embed, pos_embed, month_table):
    ts32 = timestamps.astype(jnp.int32)
    grid_spec = pltpu.PrefetchScalarGridSpec(
        num_scalar_prefetch=1,
        grid=(B // BBLK,),
        in_specs=[
            pl.BlockSpec((BBLK, T, BS, EMBED), lambda b, ts: (b, 0, 0, 0)),
            pl.BlockSpec((BS, N), lambda b, ts: (0, 0)),
            pl.BlockSpec((T, N), lambda b, ts: (0, 0)),
            pl.BlockSpec((12, N), lambda b, ts: (0, 0)),
        ],
        out_specs=pl.BlockSpec((BBLK, T, BS, EMBED), lambda b, ts: (b, 0, 0, 0)),
    )
    return pl.pallas_call(
        _body,
        grid_spec=grid_spec,
        out_shape=jax.ShapeDtypeStruct((B, T, BS, EMBED), jnp.float32),
    )(ts32, modality_tokens, channel_embed, pos_embed, month_table)


# speedup vs baseline: 2.7053x; 1.0370x over previous
import jax
import jax.numpy as jnp
from jax.experimental import pallas as pl
from jax.experimental.pallas import tpu as pltpu

B, T, BS, EMBED = 64, 24, 8, 1024
N = EMBED // 4


BBLK = 16


def _body(ts_ref, tok_ref, ch_ref, pe_ref, mt_ref, out_ref):
    bb = pl.program_id(0)
    ch = ch_ref[...]
    for bi in range(BBLK):
        b = bb * BBLK + bi
        for t in range(T):
            ts = ts_ref[b, t]
            me = mt_ref[ts, :]
            pe = pe_ref[t, :]
            tok = tok_ref[bi, t]
            out_ref[bi, t] = jnp.concatenate(
                [
                    tok[:, :N] + ch,
                    tok[:, N:2 * N] + pe[None, :],
                    tok[:, 2 * N:3 * N] + me[None, :],
                    tok[:, 3 * N:],
                ],
                axis=-1,
            )


def kernel(modality_tokens, timestamps, channel_embed, pos_embed, month_table):
    ts32 = timestamps.astype(jnp.int32)
    grid_spec = pltpu.PrefetchScalarGridSpec(
        num_scalar_prefetch=1,
        grid=(B // BBLK,),
        in_specs=[
            pl.BlockSpec((BBLK, T, BS, EMBED), lambda b, ts: (b, 0, 0, 0)),
            pl.BlockSpec((BS, N), lambda b, ts: (0, 0)),
            pl.BlockSpec((T, N), lambda b, ts: (0, 0)),
            pl.BlockSpec((12, N), lambda b, ts: (0, 0)),
        ],
        out_specs=pl.BlockSpec((BBLK, T, BS, EMBED), lambda b, ts: (b, 0, 0, 0)),
    )
    return pl.pallas_call(
        _body,
        grid_spec=grid_spec,
        out_shape=jax.ShapeDtypeStruct((B, T, BS, EMBED), jnp.float32),
    )(ts32, modality_tokens, channel_embed, pos_embed, month_table)
